# Initial kernel scaffold; baseline (speedup 1.0000x reference)
import jax
import jax.numpy as jnp
from jax.experimental import pallas as pl


def kernel(numerical, cat_0, cat_1, cat_2, cat_3, cat_4, cat_5, cat_6, cat_7, cat_8, cat_9, cat_10, cat_11, cat_12, cat_13, cat_14, cat_15, cat_16, cat_17, cat_18, cat_19, cat_20, cat_21, cat_22, cat_23, cat_24, cat_25, emb_0, emb_1, emb_2, emb_3, emb_4, emb_5, emb_6, emb_7, emb_8, emb_9, emb_10, emb_11, emb_12, emb_13, emb_14, emb_15, emb_16, emb_17, emb_18, emb_19, emb_20, emb_21, emb_22, emb_23, emb_24, emb_25, W, b):
    raise NotImplementedError("write your pallas kernel here")


# speedup vs baseline: 1.5163x; 1.5163x over previous
import functools

import jax
import jax.numpy as jnp
from jax import lax
from jax.experimental import pallas as pl
from jax.experimental.pallas import tpu as pltpu
from jax.experimental.pallas import tpu_sc as plsc

B = 16384
D = 32
NUM_TABLES = 26
NUM = 13
NUM_PAD = 16
H = 256
XW = NUM_PAD + NUM_TABLES * D

_info = plsc.get_sparse_core_info()
NC = _info.num_cores
NS = _info.num_subcores
NW = NC * NS
BPW = B // NW


def _sc_body(num_hbm, idx_hbm, *rest):
    tables = rest[:NUM_TABLES]
    x_out = rest[NUM_TABLES]
    idx_all, rows, sem_g, sem_w0, sem_w1 = rest[NUM_TABLES + 1:]

    wid = lax.axis_index("s") * NC + lax.axis_index("c")
    base = wid * BPW

    pltpu.sync_copy(idx_hbm.at[wid], idx_all)
    pltpu.sync_copy(num_hbm.at[pl.ds(base, BPW)],
                    x_out.at[pl.ds(base, BPW), pl.ds(0, NUM_PAD)])

    sem_w = (sem_w0, sem_w1)
    pending = [None, None]
    for i in range(NUM_TABLES):
        p = i % 2
        if pending[p] is not None:
            pending[p].wait()
        g = pltpu.async_copy(tables[i].at[idx_all.at[i]], rows.at[p], sem_g)
        g.wait()
        pending[p] = pltpu.async_copy(
            rows.at[p],
            x_out.at[pl.ds(base, BPW), pl.ds(NUM_PAD + D * i, D)],
            sem_w[p],
        )
    pending[0].wait()
    pending[1].wait()


_sc_fill = functools.partial(
    pl.kernel,
    mesh=plsc.VectorSubcoreMesh(core_axis_name="c", subcore_axis_name="s"),
    out_type=jax.ShapeDtypeStruct((B, XW), jnp.float32),
    compiler_params=pltpu.CompilerParams(use_tc_tiling_on_sc=False),
    scratch_types=[
        pltpu.VMEM((NUM_TABLES, BPW), jnp.int32),
        pltpu.VMEM((2, BPW, D), jnp.float32),
        pltpu.SemaphoreType.DMA,
        pltpu.SemaphoreType.DMA,
        pltpu.SemaphoreType.DMA,
    ],
)(_sc_body)


TB = 1024


def _mm_body(x_ref, w_ref, b_ref, o_ref):
    o_ref[...] = (
        jnp.dot(x_ref[...], w_ref[...], preferred_element_type=jnp.float32)
        + b_ref[...]
    )


_tc_matmul = pl.pallas_call(
    _mm_body,
    grid=(B // TB,),
    in_specs=[
        pl.BlockSpec((TB, XW), lambda i: (i, 0)),
        pl.BlockSpec((XW, H), lambda i: (0, 0)),
        pl.BlockSpec((1, H), lambda i: (0, 0)),
    ],
    out_specs=pl.BlockSpec((TB, H), lambda i: (i, 0)),
    out_shape=jax.ShapeDtypeStruct((B, H), jnp.float32),
)


def kernel(numerical, cat_0, cat_1, cat_2, cat_3, cat_4, cat_5, cat_6, cat_7, cat_8, cat_9, cat_10, cat_11, cat_12, cat_13, cat_14, cat_15, cat_16, cat_17, cat_18, cat_19, cat_20, cat_21, cat_22, cat_23, cat_24, cat_25, emb_0, emb_1, emb_2, emb_3, emb_4, emb_5, emb_6, emb_7, emb_8, emb_9, emb_10, emb_11, emb_12, emb_13, emb_14, emb_15, emb_16, emb_17, emb_18, emb_19, emb_20, emb_21, emb_22, emb_23, emb_24, emb_25, W, b):
    cats = jnp.stack(
        [cat_0, cat_1, cat_2, cat_3, cat_4, cat_5, cat_6, cat_7, cat_8,
         cat_9, cat_10, cat_11, cat_12, cat_13, cat_14, cat_15, cat_16,
         cat_17, cat_18, cat_19, cat_20, cat_21, cat_22, cat_23, cat_24,
         cat_25], axis=0).astype(jnp.int32)
    idx = cats.reshape(NUM_TABLES, NW, BPW).transpose(1, 0, 2)
    num_pad = jnp.pad(numerical, ((0, 0), (0, NUM_PAD - NUM)))
    X = _sc_fill(
        num_pad, idx,
        emb_0, emb_1, emb_2, emb_3, emb_4, emb_5, emb_6, emb_7, emb_8,
        emb_9, emb_10, emb_11, emb_12, emb_13, emb_14, emb_15, emb_16,
        emb_17, emb_18, emb_19, emb_20, emb_21, emb_22, emb_23, emb_24,
        emb_25)
    W_pad = jnp.concatenate(
        [W[:NUM], jnp.zeros((NUM_PAD - NUM, H), W.dtype), W[NUM:]], axis=0)
    return _tc_matmul(X, W_pad, b.reshape(1, H))
